# baseline (speedup 1.0000x reference)
import functools

import jax
import jax.numpy as jnp
from jax import lax
from jax.experimental import pallas as pl
from jax.experimental.pallas import tpu as pltpu
from jax.experimental.pallas import tpu_sc as plsc

B = 16384
D = 64
L = 16

_info = plsc.get_sparse_core_info()
NC, NS = _info.num_cores, _info.num_subcores
NW = NC * NS
BPW = B // NW
NCHUNK = 4
CHUNK = BPW // NCHUNK
NBLK = BPW // L


def _sc_body(idx_hbm, w_hbm, v_hbm, u_hbm, out_hbm,
             idx_v, w_rows, v_rows, u_rows, out_v, sem):
    wid = lax.axis_index("s") * NC + lax.axis_index("c")

    pltpu.sync_copy(idx_hbm.at[wid], idx_v)

    handles = []
    for t, (tab, rows) in enumerate(
            ((w_hbm, w_rows), (v_hbm, v_rows), (u_hbm, u_rows))):
        for c in range(NCHUNK):
            handles.append(pltpu.async_copy(
                tab.at[idx_v.at[t, c]],
                rows.at[pl.ds(c * CHUNK, CHUNK), :],
                sem))
    for h in handles:
        h.wait()

    lane = jnp.arange(L, dtype=jnp.int32)

    def blk_body(blk, carry):
        base = blk * L
        thetas = jnp.zeros((L,), jnp.float32)
        for r in range(L):
            row = base + r
            acc = jnp.zeros((L,), jnp.float32)
            for c in range(D // L):
                sl = pl.ds(c * L, L)
                acc = acc + w_rows[row, sl] * v_rows[row, sl] * u_rows[row, sl]
            theta = jnp.sum(acc)
            thetas = thetas + jnp.where(lane == r, theta, 0.0)
        probs = 1.0 / (1.0 + jnp.exp(-thetas))
        out_v[pl.ds(base, L)] = probs
        return carry

    lax.fori_loop(0, NBLK, blk_body, 0)

    pltpu.sync_copy(out_v, out_hbm.at[pl.ds(wid * BPW, BPW)])


@functools.partial(jax.jit, static_argnums=())
def kernel(indices, W, V, U):
    idx = indices.astype(jnp.int32).T
    idx = idx.reshape(3, NW, NCHUNK, CHUNK).transpose(1, 0, 2, 3)

    mesh = plsc.VectorSubcoreMesh(core_axis_name="c", subcore_axis_name="s")
    run = pl.kernel(
        _sc_body,
        mesh=mesh,
        out_type=jax.ShapeDtypeStruct((B,), jnp.float32),
        scratch_types=[
            pltpu.VMEM((3, NCHUNK, CHUNK), jnp.int32),
            pltpu.VMEM((BPW, D), jnp.float32),
            pltpu.VMEM((BPW, D), jnp.float32),
            pltpu.VMEM((BPW, D), jnp.float32),
            pltpu.VMEM((BPW,), jnp.float32),
            pltpu.SemaphoreType.DMA,
        ],
        compiler_params=pltpu.CompilerParams(
            needs_layout_passes=False, use_tc_tiling_on_sc=False),
    )
    return run(idx, W, V, U)
